# baseline (speedup 1.0000x reference)
import functools

import jax
import jax.numpy as jnp
import numpy as np
from jax import lax
from jax.experimental import pallas as pl
from jax.experimental.pallas import tpu as pltpu
from jax.experimental.pallas import tpu_sc as plsc

H = 128
KNN = 32
CUTOFF = 6.0
PAD_PROT = 350
PAD_LIG = 55
B = 8
RBF_C = np.linspace(0.0, 10.0, 50)
RBF_GAMMA = 10.0
BIGF = 1e30


def _silu(x):
    return x * (1.0 / (1.0 + jnp.exp(-x)))


def _graph_body(npad, pos_ref, post_ref, idx_ref, val_ref, m_scr):
    r = pos_ref.shape[0]
    pos = pos_ref[...]
    d2 = jnp.zeros((r, npad), jnp.float32)
    for ax in range(3):
        df = pos[:, ax:ax + 1] - post_ref[ax:ax + 1, :]
        d2 = d2 + df * df
    d = jnp.sqrt(d2)
    cand = (d < CUTOFF) & (d > 0.0)
    m_scr[...] = jnp.where(cand, d, BIGF)
    colio = lax.broadcasted_iota(jnp.int32, (r, npad), 1)
    idx_cols = []
    val_cols = []
    for _ in range(KNN):
        mk = m_scr[...]
        mn = jnp.min(mk, axis=1, keepdims=True)
        amn = jnp.min(jnp.where(mk == mn, colio, jnp.int32(2 ** 30)),
                      axis=1, keepdims=True)
        idx_cols.append(amn)
        val_cols.append((mn < CUTOFF).astype(jnp.float32))
        m_scr[...] = jnp.where(colio == amn, BIGF, mk)
    idx_ref[...] = jnp.concatenate(idx_cols, axis=1)
    val_ref[...] = jnp.concatenate(val_cols, axis=1)


def _build_graph(pos128, npad):
    r = 128
    grid = npad // r
    post = pos128[:, :16].T
    idx, val = pl.pallas_call(
        functools.partial(_graph_body, npad),
        grid=(grid,),
        in_specs=[
            pl.BlockSpec((r, H), lambda i: (i, 0)),
            pl.BlockSpec((16, npad), lambda i: (0, 0)),
        ],
        out_specs=[
            pl.BlockSpec((r, KNN), lambda i: (i, 0)),
            pl.BlockSpec((r, KNN), lambda i: (i, 0)),
        ],
        out_shape=[
            jax.ShapeDtypeStruct((npad, KNN), jnp.int32),
            jax.ShapeDtypeStruct((npad, KNN), jnp.float32),
        ],
        scratch_shapes=[pltpu.VMEM((r, npad), jnp.float32)],
    )(pos128, post)
    return idx, val


def _embed_body(x_ref, win_ref, bin_ref, wb_ref, h_ref, v_ref):
    h = jnp.dot(x_ref[...], win_ref[...],
                preferred_element_type=jnp.float32) + bin_ref[...]
    h_ref[...] = h
    v_ref[...] = jnp.dot(h, wb_ref[...], preferred_element_type=jnp.float32)


def _embed(xpad, win, bin_, wb0, npad):
    r = 128
    h, v = pl.pallas_call(
        _embed_body,
        grid=(npad // r,),
        in_specs=[
            pl.BlockSpec((r, H), lambda i: (i, 0)),
            pl.BlockSpec((H, H), lambda i: (0, 0)),
            pl.BlockSpec((1, H), lambda i: (0, 0)),
            pl.BlockSpec((H, H), lambda i: (0, 0)),
        ],
        out_specs=[
            pl.BlockSpec((r, H), lambda i: (i, 0)),
            pl.BlockSpec((r, H), lambda i: (i, 0)),
        ],
        out_shape=[
            jax.ShapeDtypeStruct((npad, H), jnp.float32),
            jax.ShapeDtypeStruct((npad, H), jnp.float32),
        ],
    )(xpad, win, bin_, wb0)
    return h, v


def _gather_call(vtab, ptab, idxf):
    e = idxf.shape[0]
    nw = 32
    per_w = e // nw
    ch = min(per_w, 256)
    nch = per_w // ch
    mesh = plsc.VectorSubcoreMesh(core_axis_name="c", subcore_axis_name="s")

    @functools.partial(
        pl.kernel, mesh=mesh,
        out_type=[
            jax.ShapeDtypeStruct((e, H), jnp.float32),
            jax.ShapeDtypeStruct((e, H), jnp.float32),
        ],
        scratch_types=[
            pltpu.VMEM((per_w,), jnp.int32),
            pltpu.VMEM((ch, H), jnp.float32),
            pltpu.VMEM((ch, H), jnp.float32),
            pltpu.SemaphoreType.DMA,
            pltpu.SemaphoreType.DMA,
        ],
    )
    def gk(vtab_h, ptab_h, idx_h, vg_h, pg_h, idx_v, vrows, prows,
           sem1, sem2):
        wid = lax.axis_index("s") * 2 + lax.axis_index("c")
        base = wid * per_w
        pltpu.sync_copy(idx_h.at[pl.ds(base, per_w)], idx_v)
        for ci in range(nch):
            off = base + ci * ch
            cp1 = pltpu.async_copy(
                vtab_h.at[idx_v.at[pl.ds(ci * ch, ch)]], vrows, sem1)
            cp2 = pltpu.async_copy(
                ptab_h.at[idx_v.at[pl.ds(ci * ch, ch)]], prows, sem2)
            cp1.wait()
            cp2.wait()
            pltpu.sync_copy(vrows, vg_h.at[pl.ds(off, ch)])
            pltpu.sync_copy(prows, pg_h.at[pl.ds(off, ch)])

    return gk(vtab, ptab, idxf)


def _layer_body(last, h_ref, pos_ref, vg_ref, pg_ref, val_ref,
                wa, wd, b0, e1w, e1b, c0w, c0b, c1, n0aw, n0bw, n0b,
                n1w, n1b, lng, lnb, wbn,
                ho_ref, po_ref, vo_ref):
    t = h_ref.shape[0]
    et = t * KNN
    h = h_ref[...]
    pos = pos_ref[...]
    u = jnp.dot(h, wa[...], preferred_element_type=jnp.float32) + b0[...]
    pg3 = pg_ref[...]
    diff3 = pos[:, None, :] - pg3
    d23 = jnp.sum(diff3 * diff3, axis=2, keepdims=True)
    dist3 = jnp.maximum(jnp.sqrt(jnp.maximum(d23, 1e-12)), 1e-6)
    pre3 = (u[:, None, :] + vg_ref[...]
            + dist3 * wd[...][None])
    m1 = jnp.reshape(_silu(pre3), (et, H))
    m2 = _silu(jnp.dot(m1, e1w[...], preferred_element_type=jnp.float32)
               + e1b[...])
    t0 = _silu(jnp.dot(m2, c0w[...], preferred_element_type=jnp.float32)
               + c0b[...])
    t03 = jnp.reshape(t0, (t, KNN, H))
    cw3 = jnp.sum(t03 * c1[...][None], axis=2, keepdims=True)
    cw3 = jnp.clip(cw3, -1.0, 1.0)
    val3 = val_ref[...]
    cd3 = diff3 / dist3 * (cw3 * val3)
    po_ref[...] = pos + jnp.sum(cd3, axis=1)
    m3 = jnp.reshape(m2, (t, KNN, H))
    agg = jnp.sum(m3 * val3, axis=1)
    nh = _silu(jnp.dot(h, n0aw[...], preferred_element_type=jnp.float32)
               + jnp.dot(agg, n0bw[...], preferred_element_type=jnp.float32)
               + n0b[...])
    nh = jnp.dot(nh, n1w[...], preferred_element_type=jnp.float32) + n1b[...]
    x = h + nh
    mu = jnp.mean(x, axis=1, keepdims=True)
    xc = x - mu
    var = jnp.mean(xc * xc, axis=1, keepdims=True)
    ho = xc / jnp.sqrt(var + 1e-5) * lng[...] + lnb[...]
    ho_ref[...] = ho
    if last:
        vo_ref[...] = ho
    else:
        vo_ref[...] = jnp.dot(ho, wbn[...], preferred_element_type=jnp.float32)


def _layer(p, pr, h, pos128, vg, pg, val, npad, wbn, last):
    t = 128
    grid = npad // t
    e0w = p[pr + '_edge0_w']
    wa = e0w[:H]
    wd = e0w[2 * H:2 * H + 1]
    n0w = p[pr + '_node0_w']
    args = (
        h, pos128,
        vg.reshape(npad, KNN, H),
        pg.reshape(npad, KNN, H),
        val.reshape(npad, KNN, 1),
        wa, wd, p[pr + '_edge0_b'].reshape(1, H),
        p[pr + '_edge1_w'], p[pr + '_edge1_b'].reshape(1, H),
        p[pr + '_coord0_w'], p[pr + '_coord0_b'].reshape(1, H),
        p[pr + '_coord1_w'].T,
        n0w[:H], n0w[H:], p[pr + '_node0_b'].reshape(1, H),
        p[pr + '_node1_w'], p[pr + '_node1_b'].reshape(1, H),
        p[pr + '_ln_g'].reshape(1, H), p[pr + '_ln_b'].reshape(1, H),
        wbn,
    )
    cnst = lambda i: (0, 0)
    ho, po, vo = pl.pallas_call(
        functools.partial(_layer_body, last),
        grid=(grid,),
        in_specs=[
            pl.BlockSpec((t, H), lambda i: (i, 0)),
            pl.BlockSpec((t, H), lambda i: (i, 0)),
            pl.BlockSpec((t, KNN, H), lambda i: (i, 0, 0)),
            pl.BlockSpec((t, KNN, H), lambda i: (i, 0, 0)),
            pl.BlockSpec((t, KNN, 1), lambda i: (i, 0, 0)),
            pl.BlockSpec((H, H), cnst),
            pl.BlockSpec((1, H), cnst),
            pl.BlockSpec((1, H), cnst),
            pl.BlockSpec((H, H), cnst),
            pl.BlockSpec((1, H), cnst),
            pl.BlockSpec((H, H), cnst),
            pl.BlockSpec((1, H), cnst),
            pl.BlockSpec((1, H), cnst),
            pl.BlockSpec((H, H), cnst),
            pl.BlockSpec((H, H), cnst),
            pl.BlockSpec((1, H), cnst),
            pl.BlockSpec((H, H), cnst),
            pl.BlockSpec((1, H), cnst),
            pl.BlockSpec((1, H), cnst),
            pl.BlockSpec((1, H), cnst),
            pl.BlockSpec((H, H), cnst),
        ],
        out_specs=[
            pl.BlockSpec((t, H), lambda i: (i, 0)),
            pl.BlockSpec((t, H), lambda i: (i, 0)),
            pl.BlockSpec((t, H), lambda i: (i, 0)),
        ],
        out_shape=[
            jax.ShapeDtypeStruct((npad, H), jnp.float32),
            jax.ShapeDtypeStruct((npad, H), jnp.float32),
            jax.ShapeDtypeStruct((npad, H), jnp.float32),
        ],
    )(*args)
    return ho, po, vo


def _encoder(p, enc, xflat, posflat, maskflat, n, npad):
    xpad = jnp.zeros((npad, H), jnp.float32).at[:n, :18].set(xflat)
    posm = jnp.where(maskflat[:, None] > 0, posflat, 1e6)
    pos128 = jnp.zeros((npad, H), jnp.float32)
    pos128 = pos128.at[:n, :3].set(posm)
    pos128 = pos128.at[n:, :3].set(1e9)
    idx, val = _build_graph(pos128, npad)
    idxf = idx.reshape(npad * KNN)
    win = jnp.zeros((H, H), jnp.float32).at[:18].set(p[enc + '_in_w'])
    h, v = _embed(xpad, win, p[enc + '_in_b'].reshape(1, H),
                  p[enc + '_l0_edge0_w'][H:2 * H], npad)
    pos = pos128
    for l in range(4):
        pr = f'{enc}_l{l}'
        vg, pg = _gather_call(v, pos, idxf)
        last = l == 3
        wbn = (p[f'{enc}_l{l + 1}_edge0_w'][H:2 * H] if not last
               else jnp.zeros((H, H), jnp.float32))
        h, pos, v = _layer(p, pr, h, pos, vg, pg, val, npad, wbn, last)
    return h, pos


def _attn_body(lh_ref, ph_ref, lpe_ref, ppet_ref, pmask_ref, lmask_ref,
               wq, bq, wk, bk, wv, bv, wxo, bxo, wrbft, misc,
               aff0w, aff0b, conf0w, conf0b, out_ref):
    lp = lh_ref.shape[1]
    pp = ph_ref.shape[1]
    lh = lh_ref[0]
    ph = ph_ref[0]
    lpe = lpe_ref[0]
    ppet = ppet_ref[0]
    q = jnp.dot(lh, wq[...], preferred_element_type=jnp.float32) + bq[...]
    k = jnp.dot(ph, wk[...], preferred_element_type=jnp.float32) + bk[...]
    v = jnp.dot(ph, wv[...], preferred_element_type=jnp.float32) + bv[...]
    cd2 = jnp.zeros((lp, pp), jnp.float32)
    for ax in range(3):
        df = lpe[:, ax:ax + 1] - ppet[ax:ax + 1, :]
        cd2 = cd2 + df * df
    cd = jnp.sqrt(jnp.maximum(cd2, 1e-12))
    wr = wrbft[...]
    mi = misc[...]
    biases = [jnp.zeros((lp, pp), jnp.float32) for _ in range(4)]
    for c in range(50):
        ec = jnp.exp(-RBF_GAMMA * (cd - RBF_C[c]) ** 2)
        for hh in range(4):
            biases[hh] = biases[hh] + ec * wr[hh:hh + 1, c:c + 1]
    pmask = pmask_ref[0]
    d = H // 4
    scale = 1.0 / np.sqrt(d)
    outs = []
    for hh in range(4):
        qh = q[:, hh * d:(hh + 1) * d]
        kh = k[:, hh * d:(hh + 1) * d]
        vh = v[:, hh * d:(hh + 1) * d]
        a = lax.dot_general(qh, kh, (((1,), (1,)), ((), ())),
                            preferred_element_type=jnp.float32) * scale
        a = a + biases[hh] + mi[4:5, hh:hh + 1]
        a = jnp.where(pmask == 0.0, -1e9, a)
        a = a - jnp.max(a, axis=1, keepdims=True)
        a = jnp.exp(a)
        a = a / jnp.sum(a, axis=1, keepdims=True)
        outs.append(lax.dot_general(a, vh, (((1,), (0,)), ((), ())),
                                    preferred_element_type=jnp.float32))
    out = jnp.concatenate(outs, axis=1)
    out = jnp.dot(out, wxo[...], preferred_element_type=jnp.float32) + bxo[...]
    x = lh + out
    mu = jnp.mean(x, axis=1, keepdims=True)
    xc = x - mu
    var = jnp.mean(xc * xc, axis=1, keepdims=True)
    latt = xc / jnp.sqrt(var + 1e-5) * mi[5:6, :] + mi[6:7, :]
    lm = lmask_ref[0][:, 0:1]
    n = jnp.maximum(jnp.sum(lm, axis=0, keepdims=True), 1.0)
    cr = jnp.sum(latt * lm, axis=0, keepdims=True) / n
    a0 = _silu(jnp.dot(cr, aff0w[...], preferred_element_type=jnp.float32)
               + aff0b[...])
    pkd = jnp.sum(a0 * mi[0:1, :], axis=1, keepdims=True) + mi[1:2, 0:1]
    c0 = _silu(jnp.dot(cr, conf0w[...], preferred_element_type=jnp.float32)
               + conf0b[...])
    cf = jnp.sum(c0 * mi[2:3, :H // 2], axis=1, keepdims=True) + mi[3:4, 0:1]
    cf = 1.0 / (1.0 + jnp.exp(-cf))
    out_ref[...] = (pkd * cf)[None]


def _cross_attn(p, lh_b, ph_b, lpe_b, ppet_b, pmask, lmask_b):
    lp = lh_b.shape[1]
    pp = ph_b.shape[1]
    misc = jnp.zeros((8, H), jnp.float32)
    misc = misc.at[0, :].set(p['aff1_w'][:, 0])
    misc = misc.at[1, 0].set(p['aff1_b'][0])
    misc = misc.at[2, :H // 2].set(p['conf1_w'][:, 0])
    misc = misc.at[3, 0].set(p['conf1_b'][0])
    misc = misc.at[4, :4].set(p['rbf_b'])
    misc = misc.at[5, :].set(p['xln_g'])
    misc = misc.at[6, :].set(p['xln_b'])
    wrbft = jnp.zeros((8, 56), jnp.float32).at[:4, :50].set(p['rbf_w'].T)
    cnst = lambda i: (0, 0)
    args = (lh_b, ph_b, lpe_b, ppet_b, pmask.reshape(B, 1, pp), lmask_b,
            p['q_w'], p['q_b'].reshape(1, H),
            p['k_w'], p['k_b'].reshape(1, H),
            p['v_w'], p['v_b'].reshape(1, H),
            p['xo_w'], p['xo_b'].reshape(1, H),
            wrbft, misc,
            p['aff0_w'], p['aff0_b'].reshape(1, H),
            p['conf0_w'], p['conf0_b'].reshape(1, H // 2))
    out = pl.pallas_call(
        _attn_body,
        grid=(B,),
        in_specs=[
            pl.BlockSpec((1, lp, H), lambda i: (i, 0, 0)),
            pl.BlockSpec((1, pp, H), lambda i: (i, 0, 0)),
            pl.BlockSpec((1, lp, 16), lambda i: (i, 0, 0)),
            pl.BlockSpec((1, 16, pp), lambda i: (i, 0, 0)),
            pl.BlockSpec((1, 1, pp), lambda i: (i, 0, 0)),
            pl.BlockSpec((1, lp, 8), lambda i: (i, 0, 0)),
            pl.BlockSpec((H, H), cnst),
            pl.BlockSpec((1, H), cnst),
            pl.BlockSpec((H, H), cnst),
            pl.BlockSpec((1, H), cnst),
            pl.BlockSpec((H, H), cnst),
            pl.BlockSpec((1, H), cnst),
            pl.BlockSpec((H, H), cnst),
            pl.BlockSpec((1, H), cnst),
            pl.BlockSpec((8, 56), cnst),
            pl.BlockSpec((8, H), cnst),
            pl.BlockSpec((H, H), cnst),
            pl.BlockSpec((1, H), cnst),
            pl.BlockSpec((H, H // 2), cnst),
            pl.BlockSpec((1, H // 2), cnst),
        ],
        out_specs=pl.BlockSpec((1, 1, 1), lambda i: (i, 0, 0)),
        out_shape=jax.ShapeDtypeStruct((B, 1, 1), jnp.float32),
    )(*args)
    return out[:, 0, 0]


def kernel(prot_x, prot_pos, prot_mask, lig_x, lig_pos, lig_mask, params):
    p = params
    np_prot = 2816
    np_lig = 512
    ph, ppe = _encoder(p, 'prot',
                       prot_x.reshape(B * PAD_PROT, 18),
                       prot_pos.reshape(B * PAD_PROT, 3),
                       prot_mask.reshape(B * PAD_PROT),
                       B * PAD_PROT, np_prot)
    lh, lpe = _encoder(p, 'lig',
                       lig_x.reshape(B * PAD_LIG, 18),
                       lig_pos.reshape(B * PAD_LIG, 3),
                       lig_mask.reshape(B * PAD_LIG),
                       B * PAD_LIG, np_lig)
    ph_b = ph[:B * PAD_PROT].reshape(B, PAD_PROT, H)
    ppe_b = ppe[:B * PAD_PROT, :16].reshape(B, PAD_PROT, 16)
    ppet_b = jnp.transpose(ppe_b, (0, 2, 1))
    lh_b = jnp.zeros((B, 64, H), jnp.float32).at[:, :PAD_LIG].set(
        lh[:B * PAD_LIG].reshape(B, PAD_LIG, H))
    lpe_b = jnp.zeros((B, 64, 16), jnp.float32).at[:, :PAD_LIG].set(
        lpe[:B * PAD_LIG, :16].reshape(B, PAD_LIG, 16))
    lmask_b = jnp.zeros((B, 64, 8), jnp.float32).at[:, :PAD_LIG].set(
        jnp.broadcast_to(lig_mask[:, :, None], (B, PAD_LIG, 8)))
    return _cross_attn(p, lh_b, ph_b, lpe_b, ppet_b, prot_mask, lmask_b)


# speedup vs baseline: 1.1335x; 1.1335x over previous
import functools

import jax
import jax.numpy as jnp
import numpy as np
from jax import lax
from jax.experimental import pallas as pl
from jax.experimental.pallas import tpu as pltpu
from jax.experimental.pallas import tpu_sc as plsc

H = 128
KNN = 32
CUTOFF = 6.0
PAD_PROT = 350
PAD_LIG = 55
B = 8
RBF_C = np.linspace(0.0, 10.0, 50)
RBF_GAMMA = 10.0
BIGF = 1e30


def _silu(x):
    return x * (1.0 / (1.0 + jnp.exp(-x)))


def _graph_body(npad, pos_ref, post_ref, idx_ref, val_ref, m_scr):
    r = pos_ref.shape[0]
    pos = pos_ref[...]
    d2 = jnp.zeros((r, npad), jnp.float32)
    for ax in range(3):
        df = pos[:, ax:ax + 1] - post_ref[ax:ax + 1, :]
        d2 = d2 + df * df
    d = jnp.sqrt(d2)
    cand = (d < CUTOFF) & (d > 0.0)
    m_scr[...] = jnp.where(cand, d, BIGF)
    colio = lax.broadcasted_iota(jnp.int32, (r, npad), 1)
    idx_cols = []
    val_cols = []
    for _ in range(KNN):
        mk = m_scr[...]
        mn = jnp.min(mk, axis=1, keepdims=True)
        amn = jnp.min(jnp.where(mk == mn, colio, jnp.int32(2 ** 30)),
                      axis=1, keepdims=True)
        idx_cols.append(amn)
        val_cols.append((mn < CUTOFF).astype(jnp.float32))
        m_scr[...] = jnp.where(colio == amn, BIGF, mk)
    idx_ref[...] = jnp.concatenate(idx_cols, axis=1)
    val_ref[...] = jnp.concatenate(val_cols, axis=1)


def _build_graph(pos128, npad):
    r = 128
    grid = npad // r
    post = pos128[:, :16].T
    idx, val = pl.pallas_call(
        functools.partial(_graph_body, npad),
        grid=(grid,),
        in_specs=[
            pl.BlockSpec((r, H), lambda i: (i, 0)),
            pl.BlockSpec((16, npad), lambda i: (0, 0)),
        ],
        out_specs=[
            pl.BlockSpec((r, KNN), lambda i: (i, 0)),
            pl.BlockSpec((r, KNN), lambda i: (i, 0)),
        ],
        out_shape=[
            jax.ShapeDtypeStruct((npad, KNN), jnp.int32),
            jax.ShapeDtypeStruct((npad, KNN), jnp.float32),
        ],
        scratch_shapes=[pltpu.VMEM((r, npad), jnp.float32)],
    )(pos128, post)
    return idx, val


def _embed_body(x_ref, win_ref, bin_ref, wb_ref, pos_ref, h_ref, comb_ref):
    h = jnp.dot(x_ref[...], win_ref[...],
                preferred_element_type=jnp.float32) + bin_ref[...]
    h_ref[...] = h
    v = jnp.dot(h, wb_ref[...], preferred_element_type=jnp.float32)
    comb_ref[...] = jnp.concatenate([v, pos_ref[...]], axis=1)


def _embed(xpad, win, bin_, wb0, pos128, npad):
    r = 128
    h, comb = pl.pallas_call(
        _embed_body,
        grid=(npad // r,),
        in_specs=[
            pl.BlockSpec((r, H), lambda i: (i, 0)),
            pl.BlockSpec((H, H), lambda i: (0, 0)),
            pl.BlockSpec((1, H), lambda i: (0, 0)),
            pl.BlockSpec((H, H), lambda i: (0, 0)),
            pl.BlockSpec((r, H), lambda i: (i, 0)),
        ],
        out_specs=[
            pl.BlockSpec((r, H), lambda i: (i, 0)),
            pl.BlockSpec((r, 2 * H), lambda i: (i, 0)),
        ],
        out_shape=[
            jax.ShapeDtypeStruct((npad, H), jnp.float32),
            jax.ShapeDtypeStruct((npad, 2 * H), jnp.float32),
        ],
    )(xpad, win, bin_, wb0, pos128)
    return h, comb


def _gather_call(comb, idxf):
    e = idxf.shape[0]
    nw = 32
    per_w = e // nw
    ch = per_w
    for c in (176, 128, 64):
        if per_w % c == 0:
            ch = min(per_w, c)
            break
    nch = per_w // ch
    mesh = plsc.VectorSubcoreMesh(core_axis_name="c", subcore_axis_name="s")

    @functools.partial(
        pl.kernel, mesh=mesh,
        out_type=[
            jax.ShapeDtypeStruct((e, H), jnp.float32),
            jax.ShapeDtypeStruct((e, H), jnp.float32),
        ],
        scratch_types=[
            pltpu.VMEM((per_w,), jnp.int32),
            pltpu.VMEM((ch, 2 * H), jnp.float32),
            pltpu.VMEM((ch, 2 * H), jnp.float32),
            pltpu.SemaphoreType.DMA,
            pltpu.SemaphoreType.DMA,
            pltpu.SemaphoreType.DMA,
            pltpu.SemaphoreType.DMA,
            pltpu.SemaphoreType.DMA,
            pltpu.SemaphoreType.DMA,
        ],
    )
    def gk(comb_h, idx_h, vg_h, pg_h, idx_v, buf0, buf1,
           g0, g1, ov0, ov1, op0, op1):
        wid = lax.axis_index("s") * 2 + lax.axis_index("c")
        base = wid * per_w
        bufs = (buf0, buf1)
        gsem = (g0, g1)
        ovsem = (ov0, ov1)
        opsem = (op0, op1)
        pltpu.sync_copy(idx_h.at[pl.ds(base, per_w)], idx_v)
        gh = [None, None]
        vh = [None, None]
        ph = [None, None]
        gh[0] = pltpu.async_copy(comb_h.at[idx_v.at[pl.ds(0, ch)]],
                                 bufs[0], gsem[0])
        for ci in range(nch):
            b = ci & 1
            nb = 1 - b
            if ci + 1 < nch:
                if vh[nb] is not None:
                    vh[nb].wait()
                    ph[nb].wait()
                    vh[nb] = None
                gh[nb] = pltpu.async_copy(
                    comb_h.at[idx_v.at[pl.ds((ci + 1) * ch, ch)]],
                    bufs[nb], gsem[nb])
            gh[b].wait()
            off = base + ci * ch
            vh[b] = pltpu.async_copy(bufs[b].at[:, pl.ds(0, H)],
                                     vg_h.at[pl.ds(off, ch)], ovsem[b])
            ph[b] = pltpu.async_copy(bufs[b].at[:, pl.ds(H, H)],
                                     pg_h.at[pl.ds(off, ch)], opsem[b])
        for b in range(2):
            if vh[b] is not None:
                vh[b].wait()
                ph[b].wait()

    return gk(comb, idxf)


def _layer_body(last, h_ref, pos_ref, vg_ref, pg_ref, val_ref,
                wa, wd, b0, e1w, e1b, c0w, c0b, c1, n0aw, n0bw, n0b,
                n1w, n1b, lng, lnb, wbn,
                ho_ref, comb_ref):
    t = h_ref.shape[0]
    et = t * KNN
    h = h_ref[...]
    pos = pos_ref[...]
    u = jnp.dot(h, wa[...], preferred_element_type=jnp.float32) + b0[...]
    pg3 = pg_ref[...]
    diff3 = pos[:, None, :] - pg3
    d23 = jnp.sum(diff3 * diff3, axis=2, keepdims=True)
    dist3 = jnp.maximum(jnp.sqrt(jnp.maximum(d23, 1e-12)), 1e-6)
    pre3 = (u[:, None, :] + vg_ref[...]
            + dist3 * wd[...][None])
    m1 = jnp.reshape(_silu(pre3), (et, H))
    m2 = _silu(jnp.dot(m1, e1w[...], preferred_element_type=jnp.float32)
               + e1b[...])
    t0 = _silu(jnp.dot(m2, c0w[...], preferred_element_type=jnp.float32)
               + c0b[...])
    t03 = jnp.reshape(t0, (t, KNN, H))
    cw3 = jnp.sum(t03 * c1[...][None], axis=2, keepdims=True)
    cw3 = jnp.clip(cw3, -1.0, 1.0)
    val3 = val_ref[...]
    cd3 = diff3 / dist3 * (cw3 * val3)
    po = pos + jnp.sum(cd3, axis=1)
    m3 = jnp.reshape(m2, (t, KNN, H))
    agg = jnp.sum(m3 * val3, axis=1)
    nh = _silu(jnp.dot(h, n0aw[...], preferred_element_type=jnp.float32)
               + jnp.dot(agg, n0bw[...], preferred_element_type=jnp.float32)
               + n0b[...])
    nh = jnp.dot(nh, n1w[...], preferred_element_type=jnp.float32) + n1b[...]
    x = h + nh
    mu = jnp.mean(x, axis=1, keepdims=True)
    xc = x - mu
    var = jnp.mean(xc * xc, axis=1, keepdims=True)
    ho = xc / jnp.sqrt(var + 1e-5) * lng[...] + lnb[...]
    ho_ref[...] = ho
    if last:
        vn = ho
    else:
        vn = jnp.dot(ho, wbn[...], preferred_element_type=jnp.float32)
    comb_ref[...] = jnp.concatenate([vn, po], axis=1)


def _layer(p, pr, h, comb_prev, vg, pg, val, npad, wbn, last):
    t = 128
    grid = npad // t
    e0w = p[pr + '_edge0_w']
    wa = e0w[:H]
    wd = e0w[2 * H:2 * H + 1]
    n0w = p[pr + '_node0_w']
    args = (
        h, comb_prev,
        vg.reshape(npad, KNN, H),
        pg.reshape(npad, KNN, H),
        val.reshape(npad, KNN, 1),
        wa, wd, p[pr + '_edge0_b'].reshape(1, H),
        p[pr + '_edge1_w'], p[pr + '_edge1_b'].reshape(1, H),
        p[pr + '_coord0_w'], p[pr + '_coord0_b'].reshape(1, H),
        p[pr + '_coord1_w'].T,
        n0w[:H], n0w[H:], p[pr + '_node0_b'].reshape(1, H),
        p[pr + '_node1_w'], p[pr + '_node1_b'].reshape(1, H),
        p[pr + '_ln_g'].reshape(1, H), p[pr + '_ln_b'].reshape(1, H),
        wbn,
    )
    cnst = lambda i: (0, 0)
    ho, comb = pl.pallas_call(
        functools.partial(_layer_body, last),
        grid=(grid,),
        in_specs=[
            pl.BlockSpec((t, H), lambda i: (i, 0)),
            pl.BlockSpec((t, H), lambda i: (i, 1)),
            pl.BlockSpec((t, KNN, H), lambda i: (i, 0, 0)),
            pl.BlockSpec((t, KNN, H), lambda i: (i, 0, 0)),
            pl.BlockSpec((t, KNN, 1), lambda i: (i, 0, 0)),
            pl.BlockSpec((H, H), cnst),
            pl.BlockSpec((1, H), cnst),
            pl.BlockSpec((1, H), cnst),
            pl.BlockSpec((H, H), cnst),
            pl.BlockSpec((1, H), cnst),
            pl.BlockSpec((H, H), cnst),
            pl.BlockSpec((1, H), cnst),
            pl.BlockSpec((1, H), cnst),
            pl.BlockSpec((H, H), cnst),
            pl.BlockSpec((H, H), cnst),
            pl.BlockSpec((1, H), cnst),
            pl.BlockSpec((H, H), cnst),
            pl.BlockSpec((1, H), cnst),
            pl.BlockSpec((1, H), cnst),
            pl.BlockSpec((1, H), cnst),
            pl.BlockSpec((H, H), cnst),
        ],
        out_specs=[
            pl.BlockSpec((t, H), lambda i: (i, 0)),
            pl.BlockSpec((t, 2 * H), lambda i: (i, 0)),
        ],
        out_shape=[
            jax.ShapeDtypeStruct((npad, H), jnp.float32),
            jax.ShapeDtypeStruct((npad, 2 * H), jnp.float32),
        ],
    )(*args)
    return ho, comb


def _encoder(p, enc, xflat, posflat, maskflat, n, npad):
    xpad = jnp.zeros((npad, H), jnp.float32).at[:n, :18].set(xflat)
    posm = jnp.where(maskflat[:, None] > 0, posflat, 1e6)
    pos128 = jnp.zeros((npad, H), jnp.float32)
    pos128 = pos128.at[:n, :3].set(posm)
    pos128 = pos128.at[n:, :3].set(1e9)
    idx, val = _build_graph(pos128, npad)
    idxf = idx.reshape(npad * KNN)
    win = jnp.zeros((H, H), jnp.float32).at[:18].set(p[enc + '_in_w'])
    h, comb = _embed(xpad, win, p[enc + '_in_b'].reshape(1, H),
                     p[enc + '_l0_edge0_w'][H:2 * H], pos128, npad)
    for l in range(4):
        pr = f'{enc}_l{l}'
        vg, pg = _gather_call(comb, idxf)
        last = l == 3
        wbn = (p[f'{enc}_l{l + 1}_edge0_w'][H:2 * H] if not last
               else jnp.zeros((H, H), jnp.float32))
        h, comb = _layer(p, pr, h, comb, vg, pg, val, npad, wbn, last)
    return h, comb[:, H:]


def _attn_body(lh_ref, ph_ref, lpe_ref, ppet_ref, pmask_ref, lmask_ref,
               wq, bq, wk, bk, wv, bv, wxo, bxo, wrbft, misc,
               aff0w, aff0b, conf0w, conf0b, out_ref):
    lp = lh_ref.shape[1]
    pp = ph_ref.shape[1]
    lh = lh_ref[0]
    ph = ph_ref[0]
    lpe = lpe_ref[0]
    ppet = ppet_ref[0]
    q = jnp.dot(lh, wq[...], preferred_element_type=jnp.float32) + bq[...]
    k = jnp.dot(ph, wk[...], preferred_element_type=jnp.float32) + bk[...]
    v = jnp.dot(ph, wv[...], preferred_element_type=jnp.float32) + bv[...]
    cd2 = jnp.zeros((lp, pp), jnp.float32)
    for ax in range(3):
        df = lpe[:, ax:ax + 1] - ppet[ax:ax + 1, :]
        cd2 = cd2 + df * df
    cd = jnp.sqrt(jnp.maximum(cd2, 1e-12))
    wr = wrbft[...]
    mi = misc[...]
    biases = [jnp.zeros((lp, pp), jnp.float32) for _ in range(4)]
    for c in range(50):
        ec = jnp.exp(-RBF_GAMMA * (cd - RBF_C[c]) ** 2)
        for hh in range(4):
            biases[hh] = biases[hh] + ec * wr[hh:hh + 1, c:c + 1]
    pmask = pmask_ref[0]
    d = H // 4
    scale = 1.0 / np.sqrt(d)
    outs = []
    for hh in range(4):
        qh = q[:, hh * d:(hh + 1) * d]
        kh = k[:, hh * d:(hh + 1) * d]
        vh = v[:, hh * d:(hh + 1) * d]
        a = lax.dot_general(qh, kh, (((1,), (1,)), ((), ())),
                            preferred_element_type=jnp.float32) * scale
        a = a + biases[hh] + mi[4:5, hh:hh + 1]
        a = jnp.where(pmask == 0.0, -1e9, a)
        a = a - jnp.max(a, axis=1, keepdims=True)
        a = jnp.exp(a)
        a = a / jnp.sum(a, axis=1, keepdims=True)
        outs.append(lax.dot_general(a, vh, (((1,), (0,)), ((), ())),
                                    preferred_element_type=jnp.float32))
    out = jnp.concatenate(outs, axis=1)
    out = jnp.dot(out, wxo[...], preferred_element_type=jnp.float32) + bxo[...]
    x = lh + out
    mu = jnp.mean(x, axis=1, keepdims=True)
    xc = x - mu
    var = jnp.mean(xc * xc, axis=1, keepdims=True)
    latt = xc / jnp.sqrt(var + 1e-5) * mi[5:6, :] + mi[6:7, :]
    lm = lmask_ref[0][:, 0:1]
    n = jnp.maximum(jnp.sum(lm, axis=0, keepdims=True), 1.0)
    cr = jnp.sum(latt * lm, axis=0, keepdims=True) / n
    a0 = _silu(jnp.dot(cr, aff0w[...], preferred_element_type=jnp.float32)
               + aff0b[...])
    pkd = jnp.sum(a0 * mi[0:1, :], axis=1, keepdims=True) + mi[1:2, 0:1]
    c0 = _silu(jnp.dot(cr, conf0w[...], preferred_element_type=jnp.float32)
               + conf0b[...])
    cf = jnp.sum(c0 * mi[2:3, :H // 2], axis=1, keepdims=True) + mi[3:4, 0:1]
    cf = 1.0 / (1.0 + jnp.exp(-cf))
    out_ref[...] = (pkd * cf)[None]


def _cross_attn(p, lh_b, ph_b, lpe_b, ppet_b, pmask, lmask_b):
    lp = lh_b.shape[1]
    pp = ph_b.shape[1]
    misc = jnp.zeros((8, H), jnp.float32)
    misc = misc.at[0, :].set(p['aff1_w'][:, 0])
    misc = misc.at[1, 0].set(p['aff1_b'][0])
    misc = misc.at[2, :H // 2].set(p['conf1_w'][:, 0])
    misc = misc.at[3, 0].set(p['conf1_b'][0])
    misc = misc.at[4, :4].set(p['rbf_b'])
    misc = misc.at[5, :].set(p['xln_g'])
    misc = misc.at[6, :].set(p['xln_b'])
    wrbft = jnp.zeros((8, 56), jnp.float32).at[:4, :50].set(p['rbf_w'].T)
    cnst = lambda i: (0, 0)
    args = (lh_b, ph_b, lpe_b, ppet_b, pmask.reshape(B, 1, pp), lmask_b,
            p['q_w'], p['q_b'].reshape(1, H),
            p['k_w'], p['k_b'].reshape(1, H),
            p['v_w'], p['v_b'].reshape(1, H),
            p['xo_w'], p['xo_b'].reshape(1, H),
            wrbft, misc,
            p['aff0_w'], p['aff0_b'].reshape(1, H),
            p['conf0_w'], p['conf0_b'].reshape(1, H // 2))
    out = pl.pallas_call(
        _attn_body,
        grid=(B,),
        in_specs=[
            pl.BlockSpec((1, lp, H), lambda i: (i, 0, 0)),
            pl.BlockSpec((1, pp, H), lambda i: (i, 0, 0)),
            pl.BlockSpec((1, lp, 16), lambda i: (i, 0, 0)),
            pl.BlockSpec((1, 16, pp), lambda i: (i, 0, 0)),
            pl.BlockSpec((1, 1, pp), lambda i: (i, 0, 0)),
            pl.BlockSpec((1, lp, 8), lambda i: (i, 0, 0)),
            pl.BlockSpec((H, H), cnst),
            pl.BlockSpec((1, H), cnst),
            pl.BlockSpec((H, H), cnst),
            pl.BlockSpec((1, H), cnst),
            pl.BlockSpec((H, H), cnst),
            pl.BlockSpec((1, H), cnst),
            pl.BlockSpec((H, H), cnst),
            pl.BlockSpec((1, H), cnst),
            pl.BlockSpec((8, 56), cnst),
            pl.BlockSpec((8, H), cnst),
            pl.BlockSpec((H, H), cnst),
            pl.BlockSpec((1, H), cnst),
            pl.BlockSpec((H, H // 2), cnst),
            pl.BlockSpec((1, H // 2), cnst),
        ],
        out_specs=pl.BlockSpec((1, 1, 1), lambda i: (i, 0, 0)),
        out_shape=jax.ShapeDtypeStruct((B, 1, 1), jnp.float32),
    )(*args)
    return out[:, 0, 0]


def kernel(prot_x, prot_pos, prot_mask, lig_x, lig_pos, lig_mask, params):
    p = params
    np_prot = 2816
    np_lig = 512
    ph, ppe = _encoder(p, 'prot',
                       prot_x.reshape(B * PAD_PROT, 18),
                       prot_pos.reshape(B * PAD_PROT, 3),
                       prot_mask.reshape(B * PAD_PROT),
                       B * PAD_PROT, np_prot)
    lh, lpe = _encoder(p, 'lig',
                       lig_x.reshape(B * PAD_LIG, 18),
                       lig_pos.reshape(B * PAD_LIG, 3),
                       lig_mask.reshape(B * PAD_LIG),
                       B * PAD_LIG, np_lig)
    ph_b = ph[:B * PAD_PROT].reshape(B, PAD_PROT, H)
    ppe_b = ppe[:B * PAD_PROT, :16].reshape(B, PAD_PROT, 16)
    ppet_b = jnp.transpose(ppe_b, (0, 2, 1))
    lh_b = jnp.zeros((B, 64, H), jnp.float32).at[:, :PAD_LIG].set(
        lh[:B * PAD_LIG].reshape(B, PAD_LIG, H))
    lpe_b = jnp.zeros((B, 64, 16), jnp.float32).at[:, :PAD_LIG].set(
        lpe[:B * PAD_LIG, :16].reshape(B, PAD_LIG, 16))
    lmask_b = jnp.zeros((B, 64, 8), jnp.float32).at[:, :PAD_LIG].set(
        jnp.broadcast_to(lig_mask[:, :, None], (B, PAD_LIG, 8)))
    return _cross_attn(p, lh_b, ph_b, lpe_b, ppet_b, prot_mask, lmask_b)


# speedup vs baseline: 1.2877x; 1.1361x over previous
import functools

import jax
import jax.numpy as jnp
import numpy as np
from jax import lax
from jax.experimental import pallas as pl
from jax.experimental.pallas import tpu as pltpu
from jax.experimental.pallas import tpu_sc as plsc

H = 128
KNN = 32
CUTOFF = 6.0
PAD_PROT = 350
PAD_LIG = 55
B = 8
RBF_C = np.linspace(0.0, 10.0, 50)
RBF_GAMMA = 10.0
BIGF = 1e30


def _silu(x):
    return x * (1.0 / (1.0 + jnp.exp(-x)))


def _graph_body(npad, pos_ref, post_ref, idx_ref, val_ref, m_scr):
    r = pos_ref.shape[0]
    pos = pos_ref[...]
    d2 = jnp.zeros((r, npad), jnp.float32)
    for ax in range(3):
        df = pos[:, ax:ax + 1] - post_ref[ax:ax + 1, :]
        d2 = d2 + df * df
    d = jnp.sqrt(d2)
    cand = (d < CUTOFF) & (d > 0.0)
    m_scr[...] = jnp.where(cand, d, BIGF)
    colio = lax.broadcasted_iota(jnp.int32, (r, npad), 1)
    idx_cols = []
    val_cols = []
    for _ in range(KNN):
        mk = m_scr[...]
        mn = jnp.min(mk, axis=1, keepdims=True)
        amn = jnp.min(jnp.where(mk == mn, colio, jnp.int32(2 ** 30)),
                      axis=1, keepdims=True)
        idx_cols.append(amn)
        val_cols.append((mn < CUTOFF).astype(jnp.float32))
        m_scr[...] = jnp.where(colio == amn, BIGF, mk)
    idx_ref[...] = jnp.concatenate(idx_cols, axis=1)
    val_ref[...] = jnp.concatenate(val_cols, axis=1)


def _build_graph(pos128, npad):
    r = 128
    grid = npad // r
    post = pos128[:, :16].T
    idx, val = pl.pallas_call(
        functools.partial(_graph_body, npad),
        grid=(grid,),
        in_specs=[
            pl.BlockSpec((r, H), lambda i: (i, 0)),
            pl.BlockSpec((16, npad), lambda i: (0, 0)),
        ],
        out_specs=[
            pl.BlockSpec((r, KNN), lambda i: (i, 0)),
            pl.BlockSpec((r, KNN), lambda i: (i, 0)),
        ],
        out_shape=[
            jax.ShapeDtypeStruct((npad, KNN), jnp.int32),
            jax.ShapeDtypeStruct((npad, KNN), jnp.float32),
        ],
        scratch_shapes=[pltpu.VMEM((r, npad), jnp.float32)],
    )(pos128, post)
    return idx, val


def _embed_body(x_ref, win_ref, bin_ref, wb_ref, pos_ref, h_ref, comb_ref):
    h = jnp.dot(x_ref[...], win_ref[...],
                preferred_element_type=jnp.float32) + bin_ref[...]
    h_ref[...] = h
    v = jnp.dot(h, wb_ref[...], preferred_element_type=jnp.float32)
    comb_ref[...] = jnp.concatenate([v, pos_ref[...]], axis=1)


def _embed(xpad, win, bin_, wb0, pos128, npad):
    r = 128
    h, comb = pl.pallas_call(
        _embed_body,
        grid=(npad // r,),
        in_specs=[
            pl.BlockSpec((r, H), lambda i: (i, 0)),
            pl.BlockSpec((H, H), lambda i: (0, 0)),
            pl.BlockSpec((1, H), lambda i: (0, 0)),
            pl.BlockSpec((H, H), lambda i: (0, 0)),
            pl.BlockSpec((r, H), lambda i: (i, 0)),
        ],
        out_specs=[
            pl.BlockSpec((r, H), lambda i: (i, 0)),
            pl.BlockSpec((r, 2 * H), lambda i: (i, 0)),
        ],
        out_shape=[
            jax.ShapeDtypeStruct((npad, H), jnp.float32),
            jax.ShapeDtypeStruct((npad, 2 * H), jnp.float32),
        ],
    )(xpad, win, bin_, wb0, pos128)
    return h, comb


def _gather_call(comb_p, idx_p, comb_l, idx_l):
    ep = idx_p.shape[0]
    el = idx_l.shape[0]
    nw = 32
    per_p = ep // nw
    per_l = el // nw
    chp = 176
    chl = 128
    nchp = per_p // chp
    nchl = per_l // chl
    mesh = plsc.VectorSubcoreMesh(core_axis_name="c", subcore_axis_name="s")

    @functools.partial(
        pl.kernel, mesh=mesh,
        out_type=[
            jax.ShapeDtypeStruct((ep, 2 * H), jnp.float32),
            jax.ShapeDtypeStruct((el, 2 * H), jnp.float32),
        ],
        scratch_types=[
            pltpu.VMEM((per_p,), jnp.int32),
            pltpu.VMEM((per_l,), jnp.int32),
            pltpu.VMEM((chp, 2 * H), jnp.float32),
            pltpu.VMEM((chp, 2 * H), jnp.float32),
            pltpu.SemaphoreType.DMA,
            pltpu.SemaphoreType.DMA,
            pltpu.SemaphoreType.DMA,
            pltpu.SemaphoreType.DMA,
        ],
    )
    def gk(combp_h, idxp_h, combl_h, idxl_h, gp_h, gl_h,
           ivp, ivl, buf0, buf1, g0, g1, o0, o1):
        wid = lax.axis_index("s") * 2 + lax.axis_index("c")
        base_p = wid * per_p
        base_l = wid * per_l
        bufs = (buf0, buf1)
        gsem = (g0, g1)
        osem = (o0, o1)
        pltpu.sync_copy(idxp_h.at[pl.ds(base_p, per_p)], ivp)
        pltpu.sync_copy(idxl_h.at[pl.ds(base_l, per_l)], ivl)
        items = [(combp_h, ivp, ci * chp, gp_h, base_p + ci * chp, chp)
                 for ci in range(nchp)]
        items += [(combl_h, ivl, ci * chl, gl_h, base_l + ci * chl, chl)
                  for ci in range(nchl)]

        def start(i, b):
            src, iv, ioff, _, _, c = items[i]
            dst = bufs[b] if c == chp else bufs[b].at[pl.ds(0, c)]
            return pltpu.async_copy(src.at[iv.at[pl.ds(ioff, c)]], dst,
                                    gsem[b])

        gh = [None, None]
        oh = [None, None]
        gh[0] = start(0, 0)
        for i in range(len(items)):
            b = i & 1
            nb = 1 - b
            if i + 1 < len(items):
                if oh[nb] is not None:
                    oh[nb].wait()
                    oh[nb] = None
                gh[nb] = start(i + 1, nb)
            gh[b].wait()
            _, _, _, dst, ooff, c = items[i]
            src = bufs[b] if c == chp else bufs[b].at[pl.ds(0, c)]
            oh[b] = pltpu.async_copy(src, dst.at[pl.ds(ooff, c)], osem[b])
        for b in range(2):
            if oh[b] is not None:
                oh[b].wait()

    return gk(comb_p, idx_p, comb_l, idx_l)


def _layer_body(last, h_ref, pos_ref, g_ref, val_ref,
                wa, wd, b0, e1w, e1b, c0w, c0b, c1, n0aw, n0bw, n0b,
                n1w, n1b, lng, lnb, wbn,
                ho_ref, comb_ref):
    t = h_ref.shape[0]
    et = t * KNN
    h = h_ref[...]
    pos = pos_ref[...]
    u = jnp.dot(h, wa[...], preferred_element_type=jnp.float32) + b0[...]
    g3 = g_ref[...]
    vg3 = g3[:, :, :H]
    pg3 = g3[:, :, H:]
    diff3 = pos[:, None, :] - pg3
    d23 = jnp.sum(diff3 * diff3, axis=2, keepdims=True)
    dist3 = jnp.maximum(jnp.sqrt(jnp.maximum(d23, 1e-12)), 1e-6)
    pre3 = (u[:, None, :] + vg3
            + dist3 * wd[...][None])
    m1 = jnp.reshape(_silu(pre3), (et, H))
    m2 = _silu(jnp.dot(m1, e1w[...], preferred_element_type=jnp.float32)
               + e1b[...])
    t0 = _silu(jnp.dot(m2, c0w[...], preferred_element_type=jnp.float32)
               + c0b[...])
    t03 = jnp.reshape(t0, (t, KNN, H))
    cw3 = jnp.sum(t03 * c1[...][None], axis=2, keepdims=True)
    cw3 = jnp.clip(cw3, -1.0, 1.0)
    val3 = val_ref[...]
    cd3 = diff3 / dist3 * (cw3 * val3)
    po = pos + jnp.sum(cd3, axis=1)
    m3 = jnp.reshape(m2, (t, KNN, H))
    agg = jnp.sum(m3 * val3, axis=1)
    nh = _silu(jnp.dot(h, n0aw[...], preferred_element_type=jnp.float32)
               + jnp.dot(agg, n0bw[...], preferred_element_type=jnp.float32)
               + n0b[...])
    nh = jnp.dot(nh, n1w[...], preferred_element_type=jnp.float32) + n1b[...]
    x = h + nh
    mu = jnp.mean(x, axis=1, keepdims=True)
    xc = x - mu
    var = jnp.mean(xc * xc, axis=1, keepdims=True)
    ho = xc / jnp.sqrt(var + 1e-5) * lng[...] + lnb[...]
    ho_ref[...] = ho
    if last:
        vn = ho
    else:
        vn = jnp.dot(ho, wbn[...], preferred_element_type=jnp.float32)
    comb_ref[...] = jnp.concatenate([vn, po], axis=1)


def _layer(p, pr, h, comb_prev, g, val, npad, wbn, last):
    t = 128
    grid = npad // t
    e0w = p[pr + '_edge0_w']
    wa = e0w[:H]
    wd = e0w[2 * H:2 * H + 1]
    n0w = p[pr + '_node0_w']
    args = (
        h, comb_prev,
        g.reshape(npad, KNN, 2 * H),
        val.reshape(npad, KNN, 1),
        wa, wd, p[pr + '_edge0_b'].reshape(1, H),
        p[pr + '_edge1_w'], p[pr + '_edge1_b'].reshape(1, H),
        p[pr + '_coord0_w'], p[pr + '_coord0_b'].reshape(1, H),
        p[pr + '_coord1_w'].T,
        n0w[:H], n0w[H:], p[pr + '_node0_b'].reshape(1, H),
        p[pr + '_node1_w'], p[pr + '_node1_b'].reshape(1, H),
        p[pr + '_ln_g'].reshape(1, H), p[pr + '_ln_b'].reshape(1, H),
        wbn,
    )
    cnst = lambda i: (0, 0)
    ho, comb = pl.pallas_call(
        functools.partial(_layer_body, last),
        grid=(grid,),
        in_specs=[
            pl.BlockSpec((t, H), lambda i: (i, 0)),
            pl.BlockSpec((t, H), lambda i: (i, 1)),
            pl.BlockSpec((t, KNN, 2 * H), lambda i: (i, 0, 0)),
            pl.BlockSpec((t, KNN, 1), lambda i: (i, 0, 0)),
            pl.BlockSpec((H, H), cnst),
            pl.BlockSpec((1, H), cnst),
            pl.BlockSpec((1, H), cnst),
            pl.BlockSpec((H, H), cnst),
            pl.BlockSpec((1, H), cnst),
            pl.BlockSpec((H, H), cnst),
            pl.BlockSpec((1, H), cnst),
            pl.BlockSpec((1, H), cnst),
            pl.BlockSpec((H, H), cnst),
            pl.BlockSpec((H, H), cnst),
            pl.BlockSpec((1, H), cnst),
            pl.BlockSpec((H, H), cnst),
            pl.BlockSpec((1, H), cnst),
            pl.BlockSpec((1, H), cnst),
            pl.BlockSpec((1, H), cnst),
            pl.BlockSpec((H, H), cnst),
        ],
        out_specs=[
            pl.BlockSpec((t, H), lambda i: (i, 0)),
            pl.BlockSpec((t, 2 * H), lambda i: (i, 0)),
        ],
        out_shape=[
            jax.ShapeDtypeStruct((npad, H), jnp.float32),
            jax.ShapeDtypeStruct((npad, 2 * H), jnp.float32),
        ],
    )(*args)
    return ho, comb


def _enc_pre(p, enc, xflat, posflat, maskflat, n, npad):
    xpad = jnp.zeros((npad, H), jnp.float32).at[:n, :18].set(xflat)
    posm = jnp.where(maskflat[:, None] > 0, posflat, 1e6)
    pos128 = jnp.zeros((npad, H), jnp.float32)
    pos128 = pos128.at[:n, :3].set(posm)
    pos128 = pos128.at[n:, :3].set(1e9)
    idx, val = _build_graph(pos128, npad)
    idxf = idx.reshape(npad * KNN)
    win = jnp.zeros((H, H), jnp.float32).at[:18].set(p[enc + '_in_w'])
    h, comb = _embed(xpad, win, p[enc + '_in_b'].reshape(1, H),
                     p[enc + '_l0_edge0_w'][H:2 * H], pos128, npad)
    return h, comb, idxf, val


def _attn_body(lh_ref, ph_ref, lpe_ref, ppet_ref, pmask_ref, lmask_ref,
               wq, bq, wk, bk, wv, bv, wxo, bxo, wrbft, misc,
               aff0w, aff0b, conf0w, conf0b, out_ref):
    lp = lh_ref.shape[1]
    pp = ph_ref.shape[1]
    lh = lh_ref[0]
    ph = ph_ref[0]
    lpe = lpe_ref[0]
    ppet = ppet_ref[0]
    q = jnp.dot(lh, wq[...], preferred_element_type=jnp.float32) + bq[...]
    k = jnp.dot(ph, wk[...], preferred_element_type=jnp.float32) + bk[...]
    v = jnp.dot(ph, wv[...], preferred_element_type=jnp.float32) + bv[...]
    cd2 = jnp.zeros((lp, pp), jnp.float32)
    for ax in range(3):
        df = lpe[:, ax:ax + 1] - ppet[ax:ax + 1, :]
        cd2 = cd2 + df * df
    cd = jnp.sqrt(jnp.maximum(cd2, 1e-12))
    wr = wrbft[...]
    mi = misc[...]
    biases = [jnp.zeros((lp, pp), jnp.float32) for _ in range(4)]
    for c in range(50):
        ec = jnp.exp(-RBF_GAMMA * (cd - RBF_C[c]) ** 2)
        for hh in range(4):
            biases[hh] = biases[hh] + ec * wr[hh:hh + 1, c:c + 1]
    pmask = pmask_ref[0]
    d = H // 4
    scale = 1.0 / np.sqrt(d)
    outs = []
    for hh in range(4):
        qh = q[:, hh * d:(hh + 1) * d]
        kh = k[:, hh * d:(hh + 1) * d]
        vh = v[:, hh * d:(hh + 1) * d]
        a = lax.dot_general(qh, kh, (((1,), (1,)), ((), ())),
                            preferred_element_type=jnp.float32) * scale
        a = a + biases[hh] + mi[4:5, hh:hh + 1]
        a = jnp.where(pmask == 0.0, -1e9, a)
        a = a - jnp.max(a, axis=1, keepdims=True)
        a = jnp.exp(a)
        a = a / jnp.sum(a, axis=1, keepdims=True)
        outs.append(lax.dot_general(a, vh, (((1,), (0,)), ((), ())),
                                    preferred_element_type=jnp.float32))
    out = jnp.concatenate(outs, axis=1)
    out = jnp.dot(out, wxo[...], preferred_element_type=jnp.float32) + bxo[...]
    x = lh + out
    mu = jnp.mean(x, axis=1, keepdims=True)
    xc = x - mu
    var = jnp.mean(xc * xc, axis=1, keepdims=True)
    latt = xc / jnp.sqrt(var + 1e-5) * mi[5:6, :] + mi[6:7, :]
    lm = lmask_ref[0][:, 0:1]
    n = jnp.maximum(jnp.sum(lm, axis=0, keepdims=True), 1.0)
    cr = jnp.sum(latt * lm, axis=0, keepdims=True) / n
    a0 = _silu(jnp.dot(cr, aff0w[...], preferred_element_type=jnp.float32)
               + aff0b[...])
    pkd = jnp.sum(a0 * mi[0:1, :], axis=1, keepdims=True) + mi[1:2, 0:1]
    c0 = _silu(jnp.dot(cr, conf0w[...], preferred_element_type=jnp.float32)
               + conf0b[...])
    cf = jnp.sum(c0 * mi[2:3, :H // 2], axis=1, keepdims=True) + mi[3:4, 0:1]
    cf = 1.0 / (1.0 + jnp.exp(-cf))
    out_ref[...] = (pkd * cf)[None]


def _cross_attn(p, lh_b, ph_b, lpe_b, ppet_b, pmask, lmask_b):
    lp = lh_b.shape[1]
    pp = ph_b.shape[1]
    misc = jnp.zeros((8, H), jnp.float32)
    misc = misc.at[0, :].set(p['aff1_w'][:, 0])
    misc = misc.at[1, 0].set(p['aff1_b'][0])
    misc = misc.at[2, :H // 2].set(p['conf1_w'][:, 0])
    misc = misc.at[3, 0].set(p['conf1_b'][0])
    misc = misc.at[4, :4].set(p['rbf_b'])
    misc = misc.at[5, :].set(p['xln_g'])
    misc = misc.at[6, :].set(p['xln_b'])
    wrbft = jnp.zeros((8, 56), jnp.float32).at[:4, :50].set(p['rbf_w'].T)
    cnst = lambda i: (0, 0)
    args = (lh_b, ph_b, lpe_b, ppet_b, pmask.reshape(B, 1, pp), lmask_b,
            p['q_w'], p['q_b'].reshape(1, H),
            p['k_w'], p['k_b'].reshape(1, H),
            p['v_w'], p['v_b'].reshape(1, H),
            p['xo_w'], p['xo_b'].reshape(1, H),
            wrbft, misc,
            p['aff0_w'], p['aff0_b'].reshape(1, H),
            p['conf0_w'], p['conf0_b'].reshape(1, H // 2))
    out = pl.pallas_call(
        _attn_body,
        grid=(B,),
        in_specs=[
            pl.BlockSpec((1, lp, H), lambda i: (i, 0, 0)),
            pl.BlockSpec((1, pp, H), lambda i: (i, 0, 0)),
            pl.BlockSpec((1, lp, 16), lambda i: (i, 0, 0)),
            pl.BlockSpec((1, 16, pp), lambda i: (i, 0, 0)),
            pl.BlockSpec((1, 1, pp), lambda i: (i, 0, 0)),
            pl.BlockSpec((1, lp, 8), lambda i: (i, 0, 0)),
            pl.BlockSpec((H, H), cnst),
            pl.BlockSpec((1, H), cnst),
            pl.BlockSpec((H, H), cnst),
            pl.BlockSpec((1, H), cnst),
            pl.BlockSpec((H, H), cnst),
            pl.BlockSpec((1, H), cnst),
            pl.BlockSpec((H, H), cnst),
            pl.BlockSpec((1, H), cnst),
            pl.BlockSpec((8, 56), cnst),
            pl.BlockSpec((8, H), cnst),
            pl.BlockSpec((H, H), cnst),
            pl.BlockSpec((1, H), cnst),
            pl.BlockSpec((H, H // 2), cnst),
            pl.BlockSpec((1, H // 2), cnst),
        ],
        out_specs=pl.BlockSpec((1, 1, 1), lambda i: (i, 0, 0)),
        out_shape=jax.ShapeDtypeStruct((B, 1, 1), jnp.float32),
    )(*args)
    return out[:, 0, 0]


def kernel(prot_x, prot_pos, prot_mask, lig_x, lig_pos, lig_mask, params):
    p = params
    np_prot = 2816
    np_lig = 512
    hp, comb_p, idxf_p, val_p = _enc_pre(
        p, 'prot', prot_x.reshape(B * PAD_PROT, 18),
        prot_pos.reshape(B * PAD_PROT, 3),
        prot_mask.reshape(B * PAD_PROT), B * PAD_PROT, np_prot)
    hl, comb_l, idxf_l, val_l = _enc_pre(
        p, 'lig', lig_x.reshape(B * PAD_LIG, 18),
        lig_pos.reshape(B * PAD_LIG, 3),
        lig_mask.reshape(B * PAD_LIG), B * PAD_LIG, np_lig)
    zero_w = jnp.zeros((H, H), jnp.float32)
    for l in range(4):
        gp, gl = _gather_call(comb_p, idxf_p, comb_l, idxf_l)
        last = l == 3
        wbn_p = (p[f'prot_l{l + 1}_edge0_w'][H:2 * H] if not last
                 else zero_w)
        wbn_l = (p[f'lig_l{l + 1}_edge0_w'][H:2 * H] if not last
                 else zero_w)
        hp, comb_p = _layer(p, f'prot_l{l}', hp, comb_p, gp, val_p,
                            np_prot, wbn_p, last)
        hl, comb_l = _layer(p, f'lig_l{l}', hl, comb_l, gl, val_l,
                            np_lig, wbn_l, last)
    ph, ppe = hp, comb_p[:, H:]
    lh, lpe = hl, comb_l[:, H:]
    ph_b = ph[:B * PAD_PROT].reshape(B, PAD_PROT, H)
    ppe_b = ppe[:B * PAD_PROT, :16].reshape(B, PAD_PROT, 16)
    ppet_b = jnp.transpose(ppe_b, (0, 2, 1))
    lh_b = jnp.zeros((B, 64, H), jnp.float32).at[:, :PAD_LIG].set(
        lh[:B * PAD_LIG].reshape(B, PAD_LIG, H))
    lpe_b = jnp.zeros((B, 64, 16), jnp.float32).at[:, :PAD_LIG].set(
        lpe[:B * PAD_LIG, :16].reshape(B, PAD_LIG, 16))
    lmask_b = jnp.zeros((B, 64, 8), jnp.float32).at[:, :PAD_LIG].set(
        jnp.broadcast_to(lig_mask[:, :, None], (B, PAD_LIG, 8)))
    return _cross_attn(p, lh_b, ph_b, lpe_b, ppet_b, prot_mask, lmask_b)


# speedup vs baseline: 1.2908x; 1.0024x over previous
import functools

import jax
import jax.numpy as jnp
import numpy as np
from jax import lax
from jax.experimental import pallas as pl
from jax.experimental.pallas import tpu as pltpu
from jax.experimental.pallas import tpu_sc as plsc

H = 128
KNN = 32
CUTOFF = 6.0
PAD_PROT = 350
PAD_LIG = 55
B = 8
RBF_C = np.linspace(0.0, 10.0, 50)
RBF_GAMMA = 10.0
BIGF = 1e30


def _silu(x):
    return x * (1.0 / (1.0 + jnp.exp(-x)))


def _graph_body(npad, pos_ref, post_ref, idx_ref, val_ref, m_scr):
    r = pos_ref.shape[0]
    pos = pos_ref[...]
    d2 = jnp.zeros((r, npad), jnp.float32)
    for ax in range(3):
        df = pos[:, ax:ax + 1] - post_ref[ax:ax + 1, :]
        d2 = d2 + df * df
    d = jnp.sqrt(d2)
    cand = (d < CUTOFF) & (d > 0.0)
    m_scr[...] = jnp.where(cand, d, BIGF)
    colio = lax.broadcasted_iota(jnp.int32, (r, npad), 1)
    idx_cols = []
    val_cols = []
    for _ in range(KNN):
        mk = m_scr[...]
        mn = jnp.min(mk, axis=1, keepdims=True)
        amn = jnp.min(jnp.where(mk == mn, colio, jnp.int32(2 ** 30)),
                      axis=1, keepdims=True)
        idx_cols.append(amn)
        val_cols.append((mn < CUTOFF).astype(jnp.float32))
        m_scr[...] = jnp.where(colio == amn, BIGF, mk)
    idx_ref[...] = jnp.concatenate(idx_cols, axis=1)
    val_ref[...] = jnp.concatenate(val_cols, axis=1)


def _build_graph(pos128, npad):
    r = 128
    grid = npad // r
    post = pos128[:, :16].T
    idx, val = pl.pallas_call(
        functools.partial(_graph_body, npad),
        grid=(grid,),
        in_specs=[
            pl.BlockSpec((r, H), lambda i: (i, 0)),
            pl.BlockSpec((16, npad), lambda i: (0, 0)),
        ],
        out_specs=[
            pl.BlockSpec((r, KNN), lambda i: (i, 0)),
            pl.BlockSpec((r, KNN), lambda i: (i, 0)),
        ],
        out_shape=[
            jax.ShapeDtypeStruct((npad, KNN), jnp.int32),
            jax.ShapeDtypeStruct((npad, KNN), jnp.float32),
        ],
        scratch_shapes=[pltpu.VMEM((r, npad), jnp.float32)],
    )(pos128, post)
    return idx, val


def _embed_body(x_ref, win_ref, bin_ref, wb_ref, pos_ref, h_ref, comb_ref):
    h = jnp.dot(x_ref[...], win_ref[...],
                preferred_element_type=jnp.float32) + bin_ref[...]
    h_ref[...] = h
    v = jnp.dot(h, wb_ref[...], preferred_element_type=jnp.float32)
    comb_ref[...] = jnp.concatenate([v, pos_ref[...]], axis=1)


def _embed(xpad, win, bin_, wb0, pos128, npad):
    r = 128
    h, comb = pl.pallas_call(
        _embed_body,
        grid=(npad // r,),
        in_specs=[
            pl.BlockSpec((r, H), lambda i: (i, 0)),
            pl.BlockSpec((H, H), lambda i: (0, 0)),
            pl.BlockSpec((1, H), lambda i: (0, 0)),
            pl.BlockSpec((H, H), lambda i: (0, 0)),
            pl.BlockSpec((r, H), lambda i: (i, 0)),
        ],
        out_specs=[
            pl.BlockSpec((r, H), lambda i: (i, 0)),
            pl.BlockSpec((r, 2 * H), lambda i: (i, 0)),
        ],
        out_shape=[
            jax.ShapeDtypeStruct((npad, H), jnp.float32),
            jax.ShapeDtypeStruct((npad, 2 * H), jnp.float32),
        ],
    )(xpad, win, bin_, wb0, pos128)
    return h, comb


def _gather_call(comb_p, idx_p, comb_l, idx_l):
    ep = idx_p.shape[0]
    el = idx_l.shape[0]
    nw = 32
    per_p = ep // nw
    per_l = el // nw
    chp = 88
    chl = 64
    nchp = per_p // chp
    nchl = per_l // chl
    mesh = plsc.VectorSubcoreMesh(core_axis_name="c", subcore_axis_name="s")

    @functools.partial(
        pl.kernel, mesh=mesh,
        out_type=[
            jax.ShapeDtypeStruct((ep, 2 * H), jnp.float32),
            jax.ShapeDtypeStruct((el, 2 * H), jnp.float32),
        ],
        scratch_types=[
            pltpu.VMEM((per_p,), jnp.int32),
            pltpu.VMEM((per_l,), jnp.int32),
            pltpu.VMEM((chp, 2 * H), jnp.float32),
            pltpu.VMEM((chp, 2 * H), jnp.float32),
            pltpu.VMEM((chp, 2 * H), jnp.float32),
            pltpu.VMEM((chp, 2 * H), jnp.float32),
            pltpu.SemaphoreType.DMA,
            pltpu.SemaphoreType.DMA,
            pltpu.SemaphoreType.DMA,
            pltpu.SemaphoreType.DMA,
            pltpu.SemaphoreType.DMA,
            pltpu.SemaphoreType.DMA,
            pltpu.SemaphoreType.DMA,
            pltpu.SemaphoreType.DMA,
        ],
    )
    def gk(combp_h, idxp_h, combl_h, idxl_h, gp_h, gl_h,
           ivp, ivl, buf0, buf1, buf2, buf3,
           g0, g1, g2, g3, o0, o1, o2, o3):
        wid = lax.axis_index("s") * 2 + lax.axis_index("c")
        base_p = wid * per_p
        base_l = wid * per_l
        bufs = (buf0, buf1, buf2, buf3)
        gsem = (g0, g1, g2, g3)
        osem = (o0, o1, o2, o3)
        pltpu.sync_copy(idxp_h.at[pl.ds(base_p, per_p)], ivp)
        pltpu.sync_copy(idxl_h.at[pl.ds(base_l, per_l)], ivl)
        items = [(combp_h, ivp, ci * chp, gp_h, base_p + ci * chp, chp)
                 for ci in range(nchp)]
        items += [(combl_h, ivl, ci * chl, gl_h, base_l + ci * chl, chl)
                  for ci in range(nchl)]

        def start(i, b):
            src, iv, ioff, _, _, c = items[i]
            dst = bufs[b] if c == chp else bufs[b].at[pl.ds(0, c)]
            return pltpu.async_copy(src.at[iv.at[pl.ds(ioff, c)]], dst,
                                    gsem[b])

        n = len(items)
        nbuf = 4
        depth = 2
        gh = [None] * nbuf
        oh = [None] * nbuf
        for j in range(min(depth, n)):
            gh[j % nbuf] = start(j, j % nbuf)
        for i in range(n):
            b = i % nbuf
            if i + depth < n:
                fb = (i + depth) % nbuf
                if oh[fb] is not None:
                    oh[fb].wait()
                    oh[fb] = None
                gh[fb] = start(i + depth, fb)
            gh[b].wait()
            _, _, _, dst, ooff, c = items[i]
            src = bufs[b] if c == chp else bufs[b].at[pl.ds(0, c)]
            oh[b] = pltpu.async_copy(src, dst.at[pl.ds(ooff, c)], osem[b])
        for b in range(nbuf):
            if oh[b] is not None:
                oh[b].wait()

    return gk(comb_p, idx_p, comb_l, idx_l)


def _layer_body(last, h_ref, pos_ref, g_ref, val_ref,
                wa, wd, b0, e1w, e1b, c0w, c0b, c1, n0aw, n0bw, n0b,
                n1w, n1b, lng, lnb, wbn,
                ho_ref, comb_ref):
    t = h_ref.shape[0]
    et = t * KNN
    h = h_ref[...]
    pos = pos_ref[...]
    u = jnp.dot(h, wa[...], preferred_element_type=jnp.float32) + b0[...]
    g3 = g_ref[...]
    vg3 = g3[:, :, :H]
    pg3 = g3[:, :, H:]
    diff3 = pos[:, None, :] - pg3
    d23 = jnp.sum(diff3 * diff3, axis=2, keepdims=True)
    dist3 = jnp.maximum(jnp.sqrt(jnp.maximum(d23, 1e-12)), 1e-6)
    pre3 = (u[:, None, :] + vg3
            + dist3 * wd[...][None])
    m1 = jnp.reshape(_silu(pre3), (et, H))
    m2 = _silu(jnp.dot(m1, e1w[...], preferred_element_type=jnp.float32)
               + e1b[...])
    t0 = _silu(jnp.dot(m2, c0w[...], preferred_element_type=jnp.float32)
               + c0b[...])
    t03 = jnp.reshape(t0, (t, KNN, H))
    cw3 = jnp.sum(t03 * c1[...][None], axis=2, keepdims=True)
    cw3 = jnp.clip(cw3, -1.0, 1.0)
    val3 = val_ref[...]
    cd3 = diff3 / dist3 * (cw3 * val3)
    po = pos + jnp.sum(cd3, axis=1)
    m3 = jnp.reshape(m2, (t, KNN, H))
    agg = jnp.sum(m3 * val3, axis=1)
    nh = _silu(jnp.dot(h, n0aw[...], preferred_element_type=jnp.float32)
               + jnp.dot(agg, n0bw[...], preferred_element_type=jnp.float32)
               + n0b[...])
    nh = jnp.dot(nh, n1w[...], preferred_element_type=jnp.float32) + n1b[...]
    x = h + nh
    mu = jnp.mean(x, axis=1, keepdims=True)
    xc = x - mu
    var = jnp.mean(xc * xc, axis=1, keepdims=True)
    ho = xc / jnp.sqrt(var + 1e-5) * lng[...] + lnb[...]
    ho_ref[...] = ho
    if last:
        vn = ho
    else:
        vn = jnp.dot(ho, wbn[...], preferred_element_type=jnp.float32)
    comb_ref[...] = jnp.concatenate([vn, po], axis=1)


def _layer(p, pr, h, comb_prev, g, val, npad, wbn, last):
    t = 128
    grid = npad // t
    e0w = p[pr + '_edge0_w']
    wa = e0w[:H]
    wd = e0w[2 * H:2 * H + 1]
    n0w = p[pr + '_node0_w']
    args = (
        h, comb_prev,
        g.reshape(npad, KNN, 2 * H),
        val.reshape(npad, KNN, 1),
        wa, wd, p[pr + '_edge0_b'].reshape(1, H),
        p[pr + '_edge1_w'], p[pr + '_edge1_b'].reshape(1, H),
        p[pr + '_coord0_w'], p[pr + '_coord0_b'].reshape(1, H),
        p[pr + '_coord1_w'].T,
        n0w[:H], n0w[H:], p[pr + '_node0_b'].reshape(1, H),
        p[pr + '_node1_w'], p[pr + '_node1_b'].reshape(1, H),
        p[pr + '_ln_g'].reshape(1, H), p[pr + '_ln_b'].reshape(1, H),
        wbn,
    )
    cnst = lambda i: (0, 0)
    ho, comb = pl.pallas_call(
        functools.partial(_layer_body, last),
        grid=(grid,),
        in_specs=[
            pl.BlockSpec((t, H), lambda i: (i, 0)),
            pl.BlockSpec((t, H), lambda i: (i, 1)),
            pl.BlockSpec((t, KNN, 2 * H), lambda i: (i, 0, 0)),
            pl.BlockSpec((t, KNN, 1), lambda i: (i, 0, 0)),
            pl.BlockSpec((H, H), cnst),
            pl.BlockSpec((1, H), cnst),
            pl.BlockSpec((1, H), cnst),
            pl.BlockSpec((H, H), cnst),
            pl.BlockSpec((1, H), cnst),
            pl.BlockSpec((H, H), cnst),
            pl.BlockSpec((1, H), cnst),
            pl.BlockSpec((1, H), cnst),
            pl.BlockSpec((H, H), cnst),
            pl.BlockSpec((H, H), cnst),
            pl.BlockSpec((1, H), cnst),
            pl.BlockSpec((H, H), cnst),
            pl.BlockSpec((1, H), cnst),
            pl.BlockSpec((1, H), cnst),
            pl.BlockSpec((1, H), cnst),
            pl.BlockSpec((H, H), cnst),
        ],
        out_specs=[
            pl.BlockSpec((t, H), lambda i: (i, 0)),
            pl.BlockSpec((t, 2 * H), lambda i: (i, 0)),
        ],
        out_shape=[
            jax.ShapeDtypeStruct((npad, H), jnp.float32),
            jax.ShapeDtypeStruct((npad, 2 * H), jnp.float32),
        ],
    )(*args)
    return ho, comb


def _enc_pre(p, enc, xflat, posflat, maskflat, n, npad):
    xpad = jnp.zeros((npad, H), jnp.float32).at[:n, :18].set(xflat)
    posm = jnp.where(maskflat[:, None] > 0, posflat, 1e6)
    pos128 = jnp.zeros((npad, H), jnp.float32)
    pos128 = pos128.at[:n, :3].set(posm)
    pos128 = pos128.at[n:, :3].set(1e9)
    idx, val = _build_graph(pos128, npad)
    idxf = idx.reshape(npad * KNN)
    win = jnp.zeros((H, H), jnp.float32).at[:18].set(p[enc + '_in_w'])
    h, comb = _embed(xpad, win, p[enc + '_in_b'].reshape(1, H),
                     p[enc + '_l0_edge0_w'][H:2 * H], pos128, npad)
    return h, comb, idxf, val


def _attn_body(lh_ref, ph_ref, lpe_ref, ppet_ref, pmask_ref, lmask_ref,
               wq, bq, wk, bk, wv, bv, wxo, bxo, wrbft, misc,
               aff0w, aff0b, conf0w, conf0b, out_ref):
    lp = lh_ref.shape[1]
    pp = ph_ref.shape[1]
    lh = lh_ref[0]
    ph = ph_ref[0]
    lpe = lpe_ref[0]
    ppet = ppet_ref[0]
    q = jnp.dot(lh, wq[...], preferred_element_type=jnp.float32) + bq[...]
    k = jnp.dot(ph, wk[...], preferred_element_type=jnp.float32) + bk[...]
    v = jnp.dot(ph, wv[...], preferred_element_type=jnp.float32) + bv[...]
    cd2 = jnp.zeros((lp, pp), jnp.float32)
    for ax in range(3):
        df = lpe[:, ax:ax + 1] - ppet[ax:ax + 1, :]
        cd2 = cd2 + df * df
    cd = jnp.sqrt(jnp.maximum(cd2, 1e-12))
    wr = wrbft[...]
    mi = misc[...]
    biases = [jnp.zeros((lp, pp), jnp.float32) for _ in range(4)]
    for c in range(50):
        ec = jnp.exp(-RBF_GAMMA * (cd - RBF_C[c]) ** 2)
        for hh in range(4):
            biases[hh] = biases[hh] + ec * wr[hh:hh + 1, c:c + 1]
    pmask = pmask_ref[0]
    d = H // 4
    scale = 1.0 / np.sqrt(d)
    outs = []
    for hh in range(4):
        qh = q[:, hh * d:(hh + 1) * d]
        kh = k[:, hh * d:(hh + 1) * d]
        vh = v[:, hh * d:(hh + 1) * d]
        a = lax.dot_general(qh, kh, (((1,), (1,)), ((), ())),
                            preferred_element_type=jnp.float32) * scale
        a = a + biases[hh] + mi[4:5, hh:hh + 1]
        a = jnp.where(pmask == 0.0, -1e9, a)
        a = a - jnp.max(a, axis=1, keepdims=True)
        a = jnp.exp(a)
        a = a / jnp.sum(a, axis=1, keepdims=True)
        outs.append(lax.dot_general(a, vh, (((1,), (0,)), ((), ())),
                                    preferred_element_type=jnp.float32))
    out = jnp.concatenate(outs, axis=1)
    out = jnp.dot(out, wxo[...], preferred_element_type=jnp.float32) + bxo[...]
    x = lh + out
    mu = jnp.mean(x, axis=1, keepdims=True)
    xc = x - mu
    var = jnp.mean(xc * xc, axis=1, keepdims=True)
    latt = xc / jnp.sqrt(var + 1e-5) * mi[5:6, :] + mi[6:7, :]
    lm = lmask_ref[0][:, 0:1]
    n = jnp.maximum(jnp.sum(lm, axis=0, keepdims=True), 1.0)
    cr = jnp.sum(latt * lm, axis=0, keepdims=True) / n
    a0 = _silu(jnp.dot(cr, aff0w[...], preferred_element_type=jnp.float32)
               + aff0b[...])
    pkd = jnp.sum(a0 * mi[0:1, :], axis=1, keepdims=True) + mi[1:2, 0:1]
    c0 = _silu(jnp.dot(cr, conf0w[...], preferred_element_type=jnp.float32)
               + conf0b[...])
    cf = jnp.sum(c0 * mi[2:3, :H // 2], axis=1, keepdims=True) + mi[3:4, 0:1]
    cf = 1.0 / (1.0 + jnp.exp(-cf))
    out_ref[...] = (pkd * cf)[None]


def _cross_attn(p, lh_b, ph_b, lpe_b, ppet_b, pmask, lmask_b):
    lp = lh_b.shape[1]
    pp = ph_b.shape[1]
    misc = jnp.zeros((8, H), jnp.float32)
    misc = misc.at[0, :].set(p['aff1_w'][:, 0])
    misc = misc.at[1, 0].set(p['aff1_b'][0])
    misc = misc.at[2, :H // 2].set(p['conf1_w'][:, 0])
    misc = misc.at[3, 0].set(p['conf1_b'][0])
    misc = misc.at[4, :4].set(p['rbf_b'])
    misc = misc.at[5, :].set(p['xln_g'])
    misc = misc.at[6, :].set(p['xln_b'])
    wrbft = jnp.zeros((8, 56), jnp.float32).at[:4, :50].set(p['rbf_w'].T)
    cnst = lambda i: (0, 0)
    args = (lh_b, ph_b, lpe_b, ppet_b, pmask.reshape(B, 1, pp), lmask_b,
            p['q_w'], p['q_b'].reshape(1, H),
            p['k_w'], p['k_b'].reshape(1, H),
            p['v_w'], p['v_b'].reshape(1, H),
            p['xo_w'], p['xo_b'].reshape(1, H),
            wrbft, misc,
            p['aff0_w'], p['aff0_b'].reshape(1, H),
            p['conf0_w'], p['conf0_b'].reshape(1, H // 2))
    out = pl.pallas_call(
        _attn_body,
        grid=(B,),
        in_specs=[
            pl.BlockSpec((1, lp, H), lambda i: (i, 0, 0)),
            pl.BlockSpec((1, pp, H), lambda i: (i, 0, 0)),
            pl.BlockSpec((1, lp, 16), lambda i: (i, 0, 0)),
            pl.BlockSpec((1, 16, pp), lambda i: (i, 0, 0)),
            pl.BlockSpec((1, 1, pp), lambda i: (i, 0, 0)),
            pl.BlockSpec((1, lp, 8), lambda i: (i, 0, 0)),
            pl.BlockSpec((H, H), cnst),
            pl.BlockSpec((1, H), cnst),
            pl.BlockSpec((H, H), cnst),
            pl.BlockSpec((1, H), cnst),
            pl.BlockSpec((H, H), cnst),
            pl.BlockSpec((1, H), cnst),
            pl.BlockSpec((H, H), cnst),
            pl.BlockSpec((1, H), cnst),
            pl.BlockSpec((8, 56), cnst),
            pl.BlockSpec((8, H), cnst),
            pl.BlockSpec((H, H), cnst),
            pl.BlockSpec((1, H), cnst),
            pl.BlockSpec((H, H // 2), cnst),
            pl.BlockSpec((1, H // 2), cnst),
        ],
        out_specs=pl.BlockSpec((1, 1, 1), lambda i: (i, 0, 0)),
        out_shape=jax.ShapeDtypeStruct((B, 1, 1), jnp.float32),
    )(*args)
    return out[:, 0, 0]


def kernel(prot_x, prot_pos, prot_mask, lig_x, lig_pos, lig_mask, params):
    p = params
    np_prot = 2816
    np_lig = 512
    hp, comb_p, idxf_p, val_p = _enc_pre(
        p, 'prot', prot_x.reshape(B * PAD_PROT, 18),
        prot_pos.reshape(B * PAD_PROT, 3),
        prot_mask.reshape(B * PAD_PROT), B * PAD_PROT, np_prot)
    hl, comb_l, idxf_l, val_l = _enc_pre(
        p, 'lig', lig_x.reshape(B * PAD_LIG, 18),
        lig_pos.reshape(B * PAD_LIG, 3),
        lig_mask.reshape(B * PAD_LIG), B * PAD_LIG, np_lig)
    zero_w = jnp.zeros((H, H), jnp.float32)
    for l in range(4):
        gp, gl = _gather_call(comb_p, idxf_p, comb_l, idxf_l)
        last = l == 3
        wbn_p = (p[f'prot_l{l + 1}_edge0_w'][H:2 * H] if not last
                 else zero_w)
        wbn_l = (p[f'lig_l{l + 1}_edge0_w'][H:2 * H] if not last
                 else zero_w)
        hp, comb_p = _layer(p, f'prot_l{l}', hp, comb_p, gp, val_p,
                            np_prot, wbn_p, last)
        hl, comb_l = _layer(p, f'lig_l{l}', hl, comb_l, gl, val_l,
                            np_lig, wbn_l, last)
    ph, ppe = hp, comb_p[:, H:]
    lh, lpe = hl, comb_l[:, H:]
    ph_b = ph[:B * PAD_PROT].reshape(B, PAD_PROT, H)
    ppe_b = ppe[:B * PAD_PROT, :16].reshape(B, PAD_PROT, 16)
    ppet_b = jnp.transpose(ppe_b, (0, 2, 1))
    lh_b = jnp.zeros((B, 64, H), jnp.float32).at[:, :PAD_LIG].set(
        lh[:B * PAD_LIG].reshape(B, PAD_LIG, H))
    lpe_b = jnp.zeros((B, 64, 16), jnp.float32).at[:, :PAD_LIG].set(
        lpe[:B * PAD_LIG, :16].reshape(B, PAD_LIG, 16))
    lmask_b = jnp.zeros((B, 64, 8), jnp.float32).at[:, :PAD_LIG].set(
        jnp.broadcast_to(lig_mask[:, :, None], (B, PAD_LIG, 8)))
    return _cross_attn(p, lh_b, ph_b, lpe_b, ppet_b, prot_mask, lmask_b)
